# baseline (speedup 1.0000x reference)
import functools

import jax
import jax.numpy as jnp
from jax import lax
from jax.experimental import pallas as pl
from jax.experimental.pallas import tpu as pltpu
from jax.experimental.pallas import tpu_sc as plsc


def _build(B, F, V):
    info = plsc.get_sparse_core_info()
    NC, NS, L = info.num_cores, info.num_subcores, info.num_lanes
    NW = NC * NS
    bpw = B // NW
    OVL = 2 * L - F

    mesh = plsc.VectorSubcoreMesh(core_axis_name="c", subcore_axis_name="s")

    @functools.partial(
        pl.kernel,
        out_type=jax.ShapeDtypeStruct((B,), jnp.float32),
        mesh=mesh,
        compiler_params=pltpu.CompilerParams(
            needs_layout_passes=False, use_tc_tiling_on_sc=True),
        scratch_types=[
            pltpu.VMEM((V,), jnp.float32),
            pltpu.VMEM((bpw, F), jnp.int32),
            pltpu.VMEM((bpw,), jnp.float32),
            pltpu.VMEM((48,), jnp.int32),
            pltpu.VMEM((L,), jnp.float32),
            pltpu.SemaphoreType.DMA,
            pltpu.SemaphoreType.DMA,
        ],
    )
    def k(x_hbm, tab_hbm, off_hbm, bias_hbm, out_hbm,
          tab_v, x_v, o_v, off_v, b_v, sem_t, sem_x):
        wid = lax.axis_index("s") * NC + lax.axis_index("c")
        cp_t = pltpu.async_copy(tab_hbm, tab_v, sem_t)
        cp_x = pltpu.async_copy(x_hbm.at[pl.ds(wid * bpw, bpw), :], x_v, sem_x)
        pltpu.sync_copy(off_hbm, off_v)
        pltpu.sync_copy(bias_hbm, b_v)

        off_lo = off_v[pl.ds(0, L)]
        off_hi = off_v[pl.ds(L, L)]
        rzv = off_v[pl.ds(2 * L, L)]
        bias_vec = b_v[...]
        iota = lax.iota(jnp.int32, L)
        keep = iota >= OVL
        last = iota == L - 1

        cp_x.wait()
        cp_t.wait()

        for r in range(bpw):
            v0 = x_v[r, pl.ds(0, L)]
            v1 = x_v[r, pl.ds(F - L, L)]
            t0 = plsc.load_gather(tab_v, [v0 + off_lo])
            t1 = plsc.load_gather(tab_v, [v1 + off_hi])
            s = t0 + jnp.where(keep, t1, 0.0) + bias_vec
            tot = plsc.cumsum(s)
            plsc.store_scatter(o_v, [rzv + r], tot, mask=last)

        pltpu.sync_copy(o_v, out_hbm.at[pl.ds(wid * bpw, bpw)])

    return k


def kernel(x, table, offsets, bias):
    B, F = x.shape
    V = table.shape[0]
    L = 16
    off32 = offsets.astype(jnp.int32)
    off_pack = jnp.concatenate(
        [off32[:L], off32[F - L:], jnp.zeros((L,), jnp.int32)])
    bias_pack = jnp.zeros((L,), jnp.float32).at[0].set(bias[0].astype(jnp.float32))
    out = _build(B, F, V)(x.astype(jnp.int32), table.reshape(-1), off_pack, bias_pack)
    return out[:, None]


# speedup vs baseline: 1.7723x; 1.7723x over previous
import functools

import jax
import jax.numpy as jnp
from jax import lax
from jax.experimental import pallas as pl
from jax.experimental.pallas import tpu as pltpu
from jax.experimental.pallas import tpu_sc as plsc


def _build(B, F, V):
    info = plsc.get_sparse_core_info()
    NC, NS, L = info.num_cores, info.num_subcores, info.num_lanes
    NW = NC * NS
    bpw = B // NW
    groups = bpw // L
    FP = 32

    mesh = plsc.VectorSubcoreMesh(core_axis_name="c", subcore_axis_name="s")

    @functools.partial(
        pl.kernel,
        out_type=jax.ShapeDtypeStruct((B,), jnp.float32),
        mesh=mesh,
        compiler_params=pltpu.CompilerParams(
            needs_layout_passes=False, use_tc_tiling_on_sc=True),
        scratch_types=[
            pltpu.VMEM((V,), jnp.float32),
            pltpu.VMEM((F, bpw), jnp.int32),
            pltpu.VMEM((bpw,), jnp.float32),
            pltpu.VMEM((FP,), jnp.int32),
            pltpu.VMEM((L,), jnp.float32),
            pltpu.SemaphoreType.DMA,
            pltpu.SemaphoreType.DMA,
        ],
    )
    def k(xt_hbm, tab_hbm, off_hbm, bias_hbm, out_hbm,
          tab_v, x_v, o_v, off_v, b_v, sem_t, sem_x):
        wid = lax.axis_index("s") * NC + lax.axis_index("c")
        cp_t = pltpu.async_copy(tab_hbm, tab_v, sem_t)
        cp_x = pltpu.async_copy(xt_hbm.at[:, pl.ds(wid * bpw, bpw)], x_v, sem_x)
        pltpu.sync_copy(off_hbm, off_v)
        pltpu.sync_copy(bias_hbm, b_v)

        bias_vec = b_v[...]
        off_vecs = [
            plsc.load_gather(off_v, [jnp.full((L,), f + 1, jnp.int32)])
            for f in range(F)
        ]

        cp_x.wait()
        cp_t.wait()

        for g in range(groups):
            acc = bias_vec
            for f in range(F):
                idx = x_v[f, pl.ds(g * L, L)] + off_vecs[f]
                acc = acc + plsc.load_gather(tab_v, [idx])
            o_v[pl.ds(g * L, L)] = acc
        pltpu.sync_copy(o_v, out_hbm.at[pl.ds(wid * bpw, bpw)])

    return k


def kernel(x, table, offsets, bias):
    B, F = x.shape
    V = table.shape[0]
    off_pad = jnp.zeros((32,), jnp.int32).at[1:F + 1].set(offsets.astype(jnp.int32))
    bias_pad = jnp.broadcast_to(bias.astype(jnp.float32), (16,))
    out = _build(B, F, V)(x.astype(jnp.int32).T, table.reshape(-1), off_pad, bias_pad)
    return out[:, None]


# speedup vs baseline: 1.9207x; 1.0837x over previous
import functools

import jax
import jax.numpy as jnp
from jax import lax
from jax.experimental import pallas as pl
from jax.experimental.pallas import tpu as pltpu
from jax.experimental.pallas import tpu_sc as plsc


def _build(B, F, V):
    info = plsc.get_sparse_core_info()
    NC, NS, L = info.num_cores, info.num_subcores, info.num_lanes
    NW = NC * NS
    bpw = B // NW
    groups = bpw // L
    FP = 32

    mesh = plsc.VectorSubcoreMesh(core_axis_name="c", subcore_axis_name="s")

    @functools.partial(
        pl.kernel,
        out_type=jax.ShapeDtypeStruct((B,), jnp.float32),
        mesh=mesh,
        compiler_params=pltpu.CompilerParams(
            needs_layout_passes=False, use_tc_tiling_on_sc=True),
        scratch_types=[
            pltpu.VMEM((V,), jnp.float32),
            pltpu.VMEM((F, bpw), jnp.int32),
            pltpu.VMEM((bpw,), jnp.float32),
            pltpu.VMEM((FP,), jnp.int32),
            pltpu.VMEM((L,), jnp.float32),
            pltpu.SemaphoreType.DMA,
            pltpu.SemaphoreType.DMA,
        ],
    )
    def k(xt_hbm, tab_hbm, off_hbm, bias_hbm, out_hbm,
          tab_v, x_v, o_v, off_v, b_v, sem_t, sem_x):
        wid = lax.axis_index("s") * NC + lax.axis_index("c")
        cp_t = pltpu.async_copy(tab_hbm, tab_v, sem_t)
        cp_x = pltpu.async_copy(xt_hbm.at[:, pl.ds(wid * bpw, bpw)], x_v, sem_x)
        pltpu.sync_copy(off_hbm, off_v)
        pltpu.sync_copy(bias_hbm, b_v)

        bias_vec = b_v[...]
        off_vecs = [
            plsc.load_gather(off_v, [jnp.full((L,), f + 1, jnp.int32)])
            for f in range(F)
        ]

        cp_x.wait()
        cp_t.wait()

        def body(g, carry):
            col = g * L
            acc = bias_vec
            for f in range(F):
                idx = x_v[f, pl.ds(col, L)] + off_vecs[f]
                acc = acc + plsc.load_gather(tab_v, [idx])
            o_v[pl.ds(col, L)] = acc
            return carry

        lax.fori_loop(0, groups, body, 0)
        pltpu.sync_copy(o_v, out_hbm.at[pl.ds(wid * bpw, bpw)])

    return k


def kernel(x, table, offsets, bias):
    B, F = x.shape
    V = table.shape[0]
    off_pad = jnp.zeros((32,), jnp.int32).at[1:F + 1].set(offsets.astype(jnp.int32))
    bias_pad = jnp.broadcast_to(bias.astype(jnp.float32), (16,))
    out = _build(B, F, V)(x.astype(jnp.int32).T, table.reshape(-1), off_pad, bias_pad)
    return out[:, None]


# speedup vs baseline: 1.9549x; 1.0178x over previous
import functools

import jax
import jax.numpy as jnp
from jax import lax
from jax.experimental import pallas as pl
from jax.experimental.pallas import tpu as pltpu
from jax.experimental.pallas import tpu_sc as plsc


def _build(B, F, V):
    info = plsc.get_sparse_core_info()
    NC, NS, L = info.num_cores, info.num_subcores, info.num_lanes
    NW = NC * NS
    bpw = B // NW
    groups = bpw // L
    FP = 32

    mesh = plsc.VectorSubcoreMesh(core_axis_name="c", subcore_axis_name="s")

    @functools.partial(
        pl.kernel,
        out_type=jax.ShapeDtypeStruct((B,), jnp.float32),
        mesh=mesh,
        compiler_params=pltpu.CompilerParams(
            needs_layout_passes=False, use_tc_tiling_on_sc=True),
        scratch_types=[
            pltpu.VMEM((V,), jnp.float32),
            pltpu.VMEM((F, bpw), jnp.int32),
            pltpu.VMEM((bpw,), jnp.float32),
            pltpu.VMEM((FP,), jnp.int32),
            pltpu.SemaphoreType.DMA,
            pltpu.SemaphoreType.DMA,
        ],
    )
    def k(xt_hbm, tab_hbm, off_hbm, out_hbm,
          tab_v, x_v, o_v, off_v, sem_t, sem_x):
        wid = lax.axis_index("s") * NC + lax.axis_index("c")
        cp_t = pltpu.async_copy(tab_hbm, tab_v, sem_t)
        cp_x = pltpu.async_copy(xt_hbm.at[:, pl.ds(wid * bpw, bpw)], x_v, sem_x)
        pltpu.sync_copy(off_hbm, off_v)

        bias_vec = plsc.bitcast(
            plsc.load_gather(off_v, [jnp.full((L,), FP - 1, jnp.int32)]),
            jnp.float32)
        off_vecs = [
            plsc.load_gather(off_v, [jnp.full((L,), f + 1, jnp.int32)])
            for f in range(F)
        ]

        cp_x.wait()
        cp_t.wait()

        def body(g, carry):
            col = g * L
            acc = bias_vec
            for f in range(F):
                idx = x_v[f, pl.ds(col, L)] + off_vecs[f]
                acc = acc + plsc.load_gather(tab_v, [idx])
            o_v[pl.ds(col, L)] = acc
            return carry

        lax.fori_loop(0, groups, body, 0, unroll=2)
        pltpu.sync_copy(o_v, out_hbm.at[pl.ds(wid * bpw, bpw)])

    return k


def kernel(x, table, offsets, bias):
    B, F = x.shape
    V = table.shape[0]
    bias_bits = jax.lax.bitcast_convert_type(bias.astype(jnp.float32), jnp.int32)
    pack = (jnp.zeros((32,), jnp.int32)
            .at[1:F + 1].set(offsets.astype(jnp.int32))
            .at[31].set(bias_bits[0]))
    out = _build(B, F, V)(x.astype(jnp.int32).T, table.reshape(-1), pack)
    return out[:, None]
